# baseline (speedup 1.0000x reference)
import functools

import jax
import jax.numpy as jnp
import numpy as np
from jax import lax
from jax.experimental import pallas as pl
from jax.experimental.pallas import tpu as pltpu, tpu_sc as plsc

N = 6400000
B = 50000
NC = 2
NS = 16
NW = NC * NS
L = 16

BPW = 1664
BP = NW * BPW
CHUNK = 2048
NCHUNKS = N // CHUNK
BIG = NCHUNKS - NW * (NCHUNKS // NW)
NCH = NCHUNKS // NW + 1
MAGIC = np.int32(0x5F3759DF)


def _wid():
    return lax.axis_index("s") * NC + lax.axis_index("c")


def _tile_span(wid):
    lo = 97 * wid + jnp.minimum(wid, BIG)
    nchunks = 97 + jnp.where(wid < BIG, 1, 0)
    return lo, nchunks


def _hi_ehi(en, hard, epsv):
    x2 = hard * hard + epsv
    i = plsc.bitcast(x2, jnp.int32)
    i = MAGIC - lax.shift_right_logical(i, 1)
    y = plsc.bitcast(i, jnp.float32)
    half_x2 = x2 * jnp.float32(0.5)
    for _ in range(2):
        y = y * (jnp.float32(1.5) - half_x2 * y * y)
    hi = y - epsv * (y * y)
    return hi, en * hi


def _k1_body(node_ref, batch_ref, eps_ref, phi_ref, pehi_ref,
             nbuf, bbuf, ebuf, acc_hi, acc_ehi, sem0, sem1):
    wid = _wid()
    lo, nchunks = _tile_span(wid)
    pltpu.sync_copy(eps_ref, ebuf)
    epsv = ebuf[...]

    @plsc.parallel_loop(0, BP // L, 1, unroll=4)
    def _zero(j):
        z = jnp.zeros((L,), jnp.float32)
        acc_hi[pl.ds(j * L, L)] = z
        acc_ehi[pl.ds(j * L, L)] = z

    sems = (sem0, sem1)

    def srcs(c):
        cc = jnp.minimum(c, nchunks - 1)
        off = (lo + cc) * CHUNK
        return (node_ref.at[pl.ds(off * 2, CHUNK * 2)],
                batch_ref.at[pl.ds(off, CHUNK)])

    def bufs(s):
        return (nbuf.at[pl.ds(s * CHUNK * 2, CHUNK * 2)],
                bbuf.at[pl.ds(s * CHUNK, CHUNK)])

    def fetch(c, s):
        nsrc, bsrc = srcs(c)
        ndst, bdst = bufs(s)
        pltpu.async_copy(nsrc, ndst, sems[s])
        pltpu.async_copy(bsrc, bdst, sems[s])

    def drain(c, s):
        nsrc, bsrc = srcs(c)
        ndst, bdst = bufs(s)
        pltpu.make_async_copy(nsrc, ndst, sems[s]).wait()
        pltpu.make_async_copy(bsrc, bdst, sems[s]).wait()

    fetch(0, 0)
    fetch(1, 1)

    lane0 = lax.iota(jnp.int32, L) == 0
    zv = jnp.zeros((L,), jnp.float32)

    def flush(cr):
        ci, vh, ve = cr

        @pl.when(ci >= 0)
        def _():
            idxv = jnp.broadcast_to(ci, (L,))
            plsc.addupdate_scatter(
                acc_hi, [idxv], jnp.broadcast_to(jnp.sum(vh), (L,)),
                mask=lane0)
            plsc.addupdate_scatter(
                acc_ehi, [idxv], jnp.broadcast_to(jnp.sum(ve), (L,)),
                mask=lane0)

    def process_chunk(s, carry):
        def inner(v, cr):
            w = s * CHUNK * 2 + ((v >> 3) << 8) + ((v & 7) << 4)
            k = s * CHUNK + (v << 4)
            en = nbuf[pl.ds(w, L)]
            hard = nbuf[pl.ds(w + 128, L)]
            hi, ehi = _hi_ehi(en, hard, epsv)
            bidx = bbuf[pl.ds(k, L)]
            ci, vh, ve = cr
            b15 = bidx[15]
            m_tail = bidx == jnp.broadcast_to(b15, (L,))
            m_closed = jnp.logical_not(m_tail)
            plsc.addupdate_scatter(acc_hi, [bidx], hi, mask=m_closed)
            plsc.addupdate_scatter(acc_ehi, [bidx], ehi, mask=m_closed)
            same = b15 == ci
            fmask = lane0 & jnp.broadcast_to(
                (ci >= 0) & jnp.logical_not(same), (L,))
            civ = jnp.broadcast_to(jnp.maximum(ci, 0), (L,))
            plsc.addupdate_scatter(
                acc_hi, [civ], jnp.broadcast_to(jnp.sum(vh), (L,)),
                mask=fmask)
            plsc.addupdate_scatter(
                acc_ehi, [civ], jnp.broadcast_to(jnp.sum(ve), (L,)),
                mask=fmask)
            samev = jnp.broadcast_to(same, (L,))
            vh = jnp.where(m_tail, hi, zv) + jnp.where(samev, vh, zv)
            ve = jnp.where(m_tail, ehi, zv) + jnp.where(samev, ve, zv)
            return (b15, vh, ve)
        return plsc.parallel_loop(
            0, CHUNK // L, 1, unroll=8, carry=carry)(inner)

    def pair_body(p, carry):
        for s in (0, 1):
            c = p * 2 + s
            drain(c, s)
            carry = lax.cond(c < nchunks,
                             functools.partial(process_chunk, s),
                             lambda cr: cr, carry)
            fetch(c + 2, s)
        return carry
    carry0 = (jnp.int32(-1), zv, zv)
    carry = lax.fori_loop(0, NCH // 2, pair_body, carry0)
    flush(carry)
    drain(NCH, 0)
    drain(NCH + 1, 1)

    pltpu.sync_copy(acc_hi, phi_ref.at[pl.ds(wid * BP, BP)])
    pltpu.sync_copy(acc_ehi, pehi_ref.at[pl.ds(wid * BP, BP)])


def _k2_body(phi_ref, pehi_ref, tc_ref, eps_ref, tmp_ref,
             sbuf_hi, sbuf_ehi, tcbuf, obuf, ebuf, sem):
    wid = _wid()
    base = wid * BPW
    pltpu.sync_copy(eps_ref, ebuf)
    epsv = ebuf[...]
    pltpu.sync_copy(tc_ref.at[pl.ds(base, BPW)], tcbuf)
    copies = []
    for t in range(NW):
        copies.append(pltpu.async_copy(
            phi_ref.at[pl.ds(t * BP + base, BPW)],
            sbuf_hi.at[pl.ds(t * BPW, BPW)], sem))
        copies.append(pltpu.async_copy(
            pehi_ref.at[pl.ds(t * BP + base, BPW)],
            sbuf_ehi.at[pl.ds(t * BPW, BPW)], sem))
    for c in copies:
        c.wait()

    def body(j, _):
        sl = pl.ds(j * L, L)
        shi = jnp.zeros((L,), jnp.float32)
        sehi = jnp.zeros((L,), jnp.float32)
        for t in range(NW):
            shi = shi + sbuf_hi[pl.ds(t * BPW + j * L, L)]
            sehi = sehi + sbuf_ehi[pl.ds(t * BPW + j * L, L)]
        obuf[sl] = (tcbuf[sl] + sehi) / (shi + epsv)
        return 0
    lax.fori_loop(0, BPW // L, body, 0)
    pltpu.sync_copy(obuf, tmp_ref.at[pl.ds(base, BPW)])


def _k3_body(node_ref, batch_ref, tmp_ref, eps_ref, out_ref,
             nbuf, bbuf, obuf, ebuf, tbuf, sem0, sem1, osem0, osem1):
    wid = _wid()
    lo, nchunks = _tile_span(wid)
    pltpu.sync_copy(eps_ref, ebuf)
    epsv = ebuf[...]
    pltpu.sync_copy(tmp_ref, tbuf)

    sems = (sem0, sem1)
    osems = (osem0, osem1)

    def srcs(c):
        cc = jnp.minimum(c, nchunks - 1)
        off = (lo + cc) * CHUNK
        return (node_ref.at[pl.ds(off * 2, CHUNK * 2)],
                batch_ref.at[pl.ds(off, CHUNK)])

    def odst(c):
        cc = jnp.minimum(c, nchunks - 1)
        return out_ref.at[pl.ds((lo + cc) * CHUNK, CHUNK)]

    def bufs(s):
        return (nbuf.at[pl.ds(s * CHUNK * 2, CHUNK * 2)],
                bbuf.at[pl.ds(s * CHUNK, CHUNK)])

    def oslot(s):
        return obuf.at[pl.ds(s * CHUNK, CHUNK)]

    def fetch(c, s):
        nsrc, bsrc = srcs(c)
        ndst, bdst = bufs(s)
        pltpu.async_copy(nsrc, ndst, sems[s])
        pltpu.async_copy(bsrc, bdst, sems[s])

    def drain(c, s):
        nsrc, bsrc = srcs(c)
        ndst, bdst = bufs(s)
        pltpu.make_async_copy(nsrc, ndst, sems[s]).wait()
        pltpu.make_async_copy(bsrc, bdst, sems[s]).wait()

    fetch(0, 0)
    fetch(1, 1)

    def pair_body(p, _):
        for s in (0, 1):
            c = p * 2 + s
            drain(c, s)

            @pl.when(p >= 1)
            def _():
                pltpu.make_async_copy(oslot(s), odst(c - 2), osems[s]).wait()

            @plsc.parallel_loop(0, CHUNK // L, 1, unroll=8)
            def inner(v):
                w = s * CHUNK * 2 + ((v >> 3) << 8) + ((v & 7) << 4)
                k = s * CHUNK + (v << 4)
                en = nbuf[pl.ds(w, L)]
                hard = nbuf[pl.ds(w + 128, L)]
                hi, ehi = _hi_ehi(en, hard, epsv)
                bidx = bbuf[pl.ds(k, L)]
                tv = plsc.load_gather(tbuf, [bidx])
                obuf[pl.ds(k, L)] = tv * hi - ehi
            pltpu.async_copy(oslot(s), odst(c), osems[s])
            fetch(c + 2, s)
        return 0
    lax.fori_loop(0, NCH // 2, pair_body, 0)
    drain(NCH, 0)
    drain(NCH + 1, 1)
    pltpu.make_async_copy(oslot(0), odst(NCH - 2), osems[0]).wait()
    pltpu.make_async_copy(oslot(1), odst(NCH - 1), osems[1]).wait()


@functools.lru_cache(maxsize=1)
def _build():
    mesh = plsc.VectorSubcoreMesh(
        core_axis_name="c", subcore_axis_name="s",
        num_cores=NC, num_subcores=NS)
    params = pltpu.CompilerParams(needs_layout_passes=False)
    k1 = pl.kernel(
        _k1_body,
        out_type=(jax.ShapeDtypeStruct((NW * BP,), jnp.float32),
                  jax.ShapeDtypeStruct((NW * BP,), jnp.float32)),
        mesh=mesh,
        compiler_params=params,
        scratch_types=[
            pltpu.VMEM((2 * CHUNK * 2,), jnp.float32),
            pltpu.VMEM((2 * CHUNK,), jnp.int32),
            pltpu.VMEM((L,), jnp.float32),
            pltpu.VMEM((BP,), jnp.float32),
            pltpu.VMEM((BP,), jnp.float32),
            pltpu.SemaphoreType.DMA,
            pltpu.SemaphoreType.DMA,
        ],
    )
    k2 = pl.kernel(
        _k2_body,
        out_type=jax.ShapeDtypeStruct((BP,), jnp.float32),
        mesh=mesh,
        compiler_params=params,
        scratch_types=[
            pltpu.VMEM((NW * BPW,), jnp.float32),
            pltpu.VMEM((NW * BPW,), jnp.float32),
            pltpu.VMEM((BPW,), jnp.float32),
            pltpu.VMEM((BPW,), jnp.float32),
            pltpu.VMEM((L,), jnp.float32),
            pltpu.SemaphoreType.DMA,
        ],
    )
    k3 = pl.kernel(
        _k3_body,
        out_type=jax.ShapeDtypeStruct((N,), jnp.float32),
        mesh=mesh,
        compiler_params=params,
        scratch_types=[
            pltpu.VMEM((2 * CHUNK * 2,), jnp.float32),
            pltpu.VMEM((2 * CHUNK,), jnp.int32),
            pltpu.VMEM((2 * CHUNK,), jnp.float32),
            pltpu.VMEM((L,), jnp.float32),
            pltpu.VMEM((BP,), jnp.float32),
            pltpu.SemaphoreType.DMA,
            pltpu.SemaphoreType.DMA,
            pltpu.SemaphoreType.DMA,
            pltpu.SemaphoreType.DMA,
        ],
    )
    return k1, k2, k3


def kernel(node_outputs, total_charge, batch, n_atoms, eps):
    nf = jnp.reshape(
        jnp.transpose(jnp.reshape(node_outputs, (N // 128, 128, 2)),
                      (0, 2, 1)), (2 * N,))
    eps16 = jnp.full((L,), eps, jnp.float32)
    tc_pad = jnp.pad(total_charge, (0, BP - B))
    k1, k2, k3 = _build()
    phi, pehi = k1(nf, batch, eps16)
    tmp = k2(phi, pehi, tc_pad, eps16)
    return k3(nf, batch, tmp, eps16)


# speedup vs baseline: 1.0673x; 1.0673x over previous
import functools

import jax
import jax.numpy as jnp
import numpy as np
from jax import lax
from jax.experimental import pallas as pl
from jax.experimental.pallas import tpu as pltpu, tpu_sc as plsc

N = 6400000
B = 50000
NC = 2
NS = 16
NW = NC * NS
L = 16

BPW = 1664
BP = NW * BPW
CHUNK = 2048
NCHUNKS = N // CHUNK
BIG = NCHUNKS - NW * (NCHUNKS // NW)
NCH = NCHUNKS // NW + 1
MAGIC = np.int32(0x5F3759DF)


def _wid():
    return lax.axis_index("s") * NC + lax.axis_index("c")


def _tile_span(wid):
    lo = 97 * wid + jnp.minimum(wid, BIG)
    nchunks = 97 + jnp.where(wid < BIG, 1, 0)
    return lo, nchunks


def _hi_ehi(en, hard, epsv):
    x2 = hard * hard + epsv
    i = plsc.bitcast(x2, jnp.int32)
    i = MAGIC - lax.shift_right_logical(i, 1)
    y = plsc.bitcast(i, jnp.float32)
    half_x2 = x2 * jnp.float32(0.5)
    for _ in range(2):
        y = y * (jnp.float32(1.5) - half_x2 * y * y)
    hi = y - epsv * (y * y)
    return hi, en * hi


def _k1_body(node_ref, batch_ref, eps_ref, phi_ref, pehi_ref,
             nbuf, bbuf, ebuf, acc_hi, acc_ehi, sem0, sem1):
    wid = _wid()
    lo, nchunks = _tile_span(wid)
    pltpu.sync_copy(eps_ref, ebuf)
    epsv = ebuf[...]

    @plsc.parallel_loop(0, BP // L, 1, unroll=4)
    def _zero(j):
        z = jnp.zeros((L,), jnp.float32)
        acc_hi[pl.ds(j * L, L)] = z
        acc_ehi[pl.ds(j * L, L)] = z

    sems = (sem0, sem1)

    def srcs(c):
        cc = jnp.minimum(c, nchunks - 1)
        off = (lo + cc) * CHUNK
        return (node_ref.at[pl.ds(off * 2, CHUNK * 2)],
                batch_ref.at[pl.ds(off, CHUNK)])

    def bufs(s):
        return (nbuf.at[pl.ds(s * CHUNK * 2, CHUNK * 2)],
                bbuf.at[pl.ds(s * CHUNK, CHUNK)])

    def fetch(c, s):
        nsrc, bsrc = srcs(c)
        ndst, bdst = bufs(s)
        pltpu.async_copy(nsrc, ndst, sems[s])
        pltpu.async_copy(bsrc, bdst, sems[s])

    def drain(c, s):
        nsrc, bsrc = srcs(c)
        ndst, bdst = bufs(s)
        pltpu.make_async_copy(nsrc, ndst, sems[s]).wait()
        pltpu.make_async_copy(bsrc, bdst, sems[s]).wait()

    fetch(0, 0)
    fetch(1, 1)

    lane0 = lax.iota(jnp.int32, L) == 0
    zv = jnp.zeros((L,), jnp.float32)

    def flush(cr):
        ci, vh, ve = cr

        @pl.when(ci >= 0)
        def _():
            idxv = jnp.broadcast_to(ci, (L,))
            plsc.addupdate_scatter(
                acc_hi, [idxv], jnp.broadcast_to(jnp.sum(vh), (L,)),
                mask=lane0)
            plsc.addupdate_scatter(
                acc_ehi, [idxv], jnp.broadcast_to(jnp.sum(ve), (L,)),
                mask=lane0)

    def process_chunk(s, carry):
        def inner(v, cr):
            w = s * CHUNK * 2 + ((v >> 3) << 8) + ((v & 7) << 4)
            k = s * CHUNK + (v << 4)
            en = nbuf[pl.ds(w, L)]
            hard = nbuf[pl.ds(w + 128, L)]
            hi, ehi = _hi_ehi(en, hard, epsv)
            bidx = bbuf[pl.ds(k, L)]
            ci, vh, ve = cr
            b15 = bidx[15]
            m_tail = bidx == jnp.broadcast_to(b15, (L,))
            m_closed = jnp.logical_not(m_tail)
            plsc.addupdate_scatter(acc_hi, [bidx], hi, mask=m_closed)
            plsc.addupdate_scatter(acc_ehi, [bidx], ehi, mask=m_closed)
            same = b15 == ci
            fmask = lane0 & jnp.broadcast_to(
                (ci >= 0) & jnp.logical_not(same), (L,))
            civ = jnp.broadcast_to(jnp.maximum(ci, 0), (L,))
            plsc.addupdate_scatter(
                acc_hi, [civ], jnp.broadcast_to(jnp.sum(vh), (L,)),
                mask=fmask)
            plsc.addupdate_scatter(
                acc_ehi, [civ], jnp.broadcast_to(jnp.sum(ve), (L,)),
                mask=fmask)
            samev = jnp.broadcast_to(same, (L,))
            vh = jnp.where(m_tail, hi, zv) + jnp.where(samev, vh, zv)
            ve = jnp.where(m_tail, ehi, zv) + jnp.where(samev, ve, zv)
            return (b15, vh, ve)
        return plsc.parallel_loop(
            0, CHUNK // L, 1, unroll=4, carry=carry)(inner)

    def pair_body(p, carry):
        for s in (0, 1):
            c = p * 2 + s
            drain(c, s)
            carry = lax.cond(c < nchunks,
                             functools.partial(process_chunk, s),
                             lambda cr: cr, carry)
            fetch(c + 2, s)
        return carry
    carry0 = (jnp.int32(-1), zv, zv)
    carry = lax.fori_loop(0, NCH // 2, pair_body, carry0)
    flush(carry)
    drain(NCH, 0)
    drain(NCH + 1, 1)

    pltpu.sync_copy(acc_hi, phi_ref.at[pl.ds(wid * BP, BP)])
    pltpu.sync_copy(acc_ehi, pehi_ref.at[pl.ds(wid * BP, BP)])


def _k2_body(phi_ref, pehi_ref, tc_ref, eps_ref, tmp_ref,
             sbuf_hi, sbuf_ehi, tcbuf, obuf, ebuf, sem):
    wid = _wid()
    base = wid * BPW
    pltpu.sync_copy(eps_ref, ebuf)
    epsv = ebuf[...]
    pltpu.sync_copy(tc_ref.at[pl.ds(base, BPW)], tcbuf)
    copies = []
    for t in range(NW):
        copies.append(pltpu.async_copy(
            phi_ref.at[pl.ds(t * BP + base, BPW)],
            sbuf_hi.at[pl.ds(t * BPW, BPW)], sem))
        copies.append(pltpu.async_copy(
            pehi_ref.at[pl.ds(t * BP + base, BPW)],
            sbuf_ehi.at[pl.ds(t * BPW, BPW)], sem))
    for c in copies:
        c.wait()

    def body(j, _):
        sl = pl.ds(j * L, L)
        shi = jnp.zeros((L,), jnp.float32)
        sehi = jnp.zeros((L,), jnp.float32)
        for t in range(NW):
            shi = shi + sbuf_hi[pl.ds(t * BPW + j * L, L)]
            sehi = sehi + sbuf_ehi[pl.ds(t * BPW + j * L, L)]
        obuf[sl] = (tcbuf[sl] + sehi) / (shi + epsv)
        return 0
    lax.fori_loop(0, BPW // L, body, 0)
    pltpu.sync_copy(obuf, tmp_ref.at[pl.ds(base, BPW)])


def _k3_body(node_ref, batch_ref, tmp_ref, eps_ref, out_ref,
             nbuf, bbuf, obuf, ebuf, tbuf, sem0, sem1, osem0, osem1):
    wid = _wid()
    lo, nchunks = _tile_span(wid)
    pltpu.sync_copy(eps_ref, ebuf)
    epsv = ebuf[...]
    pltpu.sync_copy(tmp_ref, tbuf)

    sems = (sem0, sem1)
    osems = (osem0, osem1)

    def srcs(c):
        cc = jnp.minimum(c, nchunks - 1)
        off = (lo + cc) * CHUNK
        return (node_ref.at[pl.ds(off * 2, CHUNK * 2)],
                batch_ref.at[pl.ds(off, CHUNK)])

    def odst(c):
        cc = jnp.minimum(c, nchunks - 1)
        return out_ref.at[pl.ds((lo + cc) * CHUNK, CHUNK)]

    def bufs(s):
        return (nbuf.at[pl.ds(s * CHUNK * 2, CHUNK * 2)],
                bbuf.at[pl.ds(s * CHUNK, CHUNK)])

    def oslot(s):
        return obuf.at[pl.ds(s * CHUNK, CHUNK)]

    def fetch(c, s):
        nsrc, bsrc = srcs(c)
        ndst, bdst = bufs(s)
        pltpu.async_copy(nsrc, ndst, sems[s])
        pltpu.async_copy(bsrc, bdst, sems[s])

    def drain(c, s):
        nsrc, bsrc = srcs(c)
        ndst, bdst = bufs(s)
        pltpu.make_async_copy(nsrc, ndst, sems[s]).wait()
        pltpu.make_async_copy(bsrc, bdst, sems[s]).wait()

    fetch(0, 0)
    fetch(1, 1)

    def pair_body(p, _):
        for s in (0, 1):
            c = p * 2 + s
            drain(c, s)

            @pl.when(p >= 1)
            def _():
                pltpu.make_async_copy(oslot(s), odst(c - 2), osems[s]).wait()

            @plsc.parallel_loop(0, CHUNK // L, 1, unroll=4)
            def inner(v):
                w = s * CHUNK * 2 + ((v >> 3) << 8) + ((v & 7) << 4)
                k = s * CHUNK + (v << 4)
                en = nbuf[pl.ds(w, L)]
                hard = nbuf[pl.ds(w + 128, L)]
                hi, ehi = _hi_ehi(en, hard, epsv)
                bidx = bbuf[pl.ds(k, L)]
                tv = plsc.load_gather(tbuf, [bidx])
                obuf[pl.ds(k, L)] = tv * hi - ehi
            pltpu.async_copy(oslot(s), odst(c), osems[s])
            fetch(c + 2, s)
        return 0
    lax.fori_loop(0, NCH // 2, pair_body, 0)
    drain(NCH, 0)
    drain(NCH + 1, 1)
    pltpu.make_async_copy(oslot(0), odst(NCH - 2), osems[0]).wait()
    pltpu.make_async_copy(oslot(1), odst(NCH - 1), osems[1]).wait()


@functools.lru_cache(maxsize=1)
def _build():
    mesh = plsc.VectorSubcoreMesh(
        core_axis_name="c", subcore_axis_name="s",
        num_cores=NC, num_subcores=NS)
    params = pltpu.CompilerParams(needs_layout_passes=False)
    k1 = pl.kernel(
        _k1_body,
        out_type=(jax.ShapeDtypeStruct((NW * BP,), jnp.float32),
                  jax.ShapeDtypeStruct((NW * BP,), jnp.float32)),
        mesh=mesh,
        compiler_params=params,
        scratch_types=[
            pltpu.VMEM((2 * CHUNK * 2,), jnp.float32),
            pltpu.VMEM((2 * CHUNK,), jnp.int32),
            pltpu.VMEM((L,), jnp.float32),
            pltpu.VMEM((BP,), jnp.float32),
            pltpu.VMEM((BP,), jnp.float32),
            pltpu.SemaphoreType.DMA,
            pltpu.SemaphoreType.DMA,
        ],
    )
    k2 = pl.kernel(
        _k2_body,
        out_type=jax.ShapeDtypeStruct((BP,), jnp.float32),
        mesh=mesh,
        compiler_params=params,
        scratch_types=[
            pltpu.VMEM((NW * BPW,), jnp.float32),
            pltpu.VMEM((NW * BPW,), jnp.float32),
            pltpu.VMEM((BPW,), jnp.float32),
            pltpu.VMEM((BPW,), jnp.float32),
            pltpu.VMEM((L,), jnp.float32),
            pltpu.SemaphoreType.DMA,
        ],
    )
    k3 = pl.kernel(
        _k3_body,
        out_type=jax.ShapeDtypeStruct((N,), jnp.float32),
        mesh=mesh,
        compiler_params=params,
        scratch_types=[
            pltpu.VMEM((2 * CHUNK * 2,), jnp.float32),
            pltpu.VMEM((2 * CHUNK,), jnp.int32),
            pltpu.VMEM((2 * CHUNK,), jnp.float32),
            pltpu.VMEM((L,), jnp.float32),
            pltpu.VMEM((BP,), jnp.float32),
            pltpu.SemaphoreType.DMA,
            pltpu.SemaphoreType.DMA,
            pltpu.SemaphoreType.DMA,
            pltpu.SemaphoreType.DMA,
        ],
    )
    return k1, k2, k3


def kernel(node_outputs, total_charge, batch, n_atoms, eps):
    nf = jnp.reshape(
        jnp.transpose(jnp.reshape(node_outputs, (N // 128, 128, 2)),
                      (0, 2, 1)), (2 * N,))
    eps16 = jnp.full((L,), eps, jnp.float32)
    tc_pad = jnp.pad(total_charge, (0, BP - B))
    k1, k2, k3 = _build()
    phi, pehi = k1(nf, batch, eps16)
    tmp = k2(phi, pehi, tc_pad, eps16)
    return k3(nf, batch, tmp, eps16)
